# baseline (speedup 1.0000x reference)
import jax
import jax.numpy as jnp
from jax import lax
from jax.experimental import pallas as pl
from jax.experimental.pallas import tpu as pltpu
from jax.experimental.pallas import tpu_sc as plsc

_NC = 2
_NS = 16
_NW = _NC * _NS
_L = 16
_B = 16384
_D = 32
_BPW = _B // _NW
_CH = 128
_NCHUNK = _BPW // _CH
_GPC = _CH // _L


def _gmf_body(users, movies, utab, mtab, w, b, out,
              uidx, midx, uq, mq, urows, mrows, wv, bv, outv, sem):
    wid = lax.axis_index("s") * _NC + lax.axis_index("c")
    base = wid * _BPW

    for c in range(_NCHUNK):
        pltpu.sync_copy(users.at[pl.ds(base + c * _CH, _CH)], uidx.at[c])
        pltpu.sync_copy(movies.at[pl.ds(base + c * _CH, _CH)], midx.at[c])
    pltpu.sync_copy(w, wv)
    pltpu.sync_copy(b, bv)

    for c in range(_NCHUNK):
        for g in range(_GPC):
            o = g * _L
            uq.at[c][pl.ds(o, _L)] = jnp.right_shift(uidx.at[c][pl.ds(o, _L)], 2)
            mq.at[c][pl.ds(o, _L)] = jnp.right_shift(midx.at[c][pl.ds(o, _L)], 2)

    iota = lax.iota(jnp.int32, _L)
    bias = bv[...]

    for c in range(_NCHUNK):
        cp_u = pltpu.async_copy(utab.at[uq.at[c]], urows, sem)
        cp_m = pltpu.async_copy(mtab.at[mq.at[c]], mrows, sem)
        cp_u.wait()
        cp_m.wait()

        def group_body(g, carry):
            o = g * _L
            uv = uidx.at[c][pl.ds(o, _L)]
            mv = midx.at[c][pl.ds(o, _L)]
            rows = o + iota
            ucol = jnp.left_shift(jnp.bitwise_and(uv, 3), 5)
            mcol = jnp.left_shift(jnp.bitwise_and(mv, 3), 5)
            acc0 = plsc.load_gather(urows, [rows, ucol]) \
                * plsc.load_gather(mrows, [rows, mcol]) * wv[0]
            acc1 = plsc.load_gather(urows, [rows, ucol + 1]) \
                * plsc.load_gather(mrows, [rows, mcol + 1]) * wv[1]
            for d in range(2, _D, 2):
                acc0 = acc0 + plsc.load_gather(urows, [rows, ucol + d]) \
                    * plsc.load_gather(mrows, [rows, mcol + d]) * wv[d]
                acc1 = acc1 + plsc.load_gather(urows, [rows, ucol + d + 1]) \
                    * plsc.load_gather(mrows, [rows, mcol + d + 1]) * wv[d + 1]
            x = acc0 + acc1 + bias
            y = 1.0 / (1.0 + jnp.exp(-x))
            outv[pl.ds(c * _CH + o, _L)] = y
            return carry

        lax.fori_loop(0, _GPC, group_body, 0)

    pltpu.sync_copy(outv, out.at[pl.ds(base, _BPW)])


def kernel(users, movies, user_table, movie_table, lin_w, lin_b):
    mesh = plsc.VectorSubcoreMesh(core_axis_name="c", subcore_axis_name="s")
    f = pl.kernel(
        _gmf_body,
        mesh=mesh,
        compiler_params=pltpu.CompilerParams(needs_layout_passes=False),
        out_type=jax.ShapeDtypeStruct((_B,), jnp.float32),
        scratch_types=[
            pltpu.VMEM((_NCHUNK, _CH), jnp.int32),
            pltpu.VMEM((_NCHUNK, _CH), jnp.int32),
            pltpu.VMEM((_NCHUNK, _CH), jnp.int32),
            pltpu.VMEM((_NCHUNK, _CH), jnp.int32),
            pltpu.VMEM((_CH, 128), jnp.float32),
            pltpu.VMEM((_CH, 128), jnp.float32),
            pltpu.VMEM((_D, _L), jnp.float32),
            pltpu.VMEM((_L,), jnp.float32),
            pltpu.VMEM((_BPW,), jnp.float32),
            pltpu.SemaphoreType.DMA,
        ],
    )
    wb = jnp.broadcast_to(
        lin_w.astype(jnp.float32).reshape(_D, 1), (_D, _L))
    bb = jnp.broadcast_to(lin_b.astype(jnp.float32).reshape(()), (_L,))
    out = f(users.astype(jnp.int32), movies.astype(jnp.int32),
            user_table.reshape(-1, 128), movie_table.reshape(-1, 128),
            wb, bb)
    return out.reshape(_B, 1)


# speedup vs baseline: 1.0050x; 1.0050x over previous
import jax
import jax.numpy as jnp
from jax import lax
from jax.experimental import pallas as pl
from jax.experimental.pallas import tpu as pltpu
from jax.experimental.pallas import tpu_sc as plsc

_NC = 2
_NS = 16
_NW = _NC * _NS
_L = 16
_B = 16384
_D = 32
_BPW = _B // _NW
_CH = 128
_NCHUNK = _BPW // _CH
_GPC = _CH // _L


def _gmf_body(users, movies, utab, mtab, w, b, out,
              uidx, midx, uq, mq, urows0, mrows0, urows1, mrows1,
              wv, bv, outv, sem0, sem1):
    wid = lax.axis_index("s") * _NC + lax.axis_index("c")
    base = wid * _BPW

    for c in range(_NCHUNK):
        pltpu.sync_copy(users.at[pl.ds(base + c * _CH, _CH)], uidx.at[c])
        pltpu.sync_copy(movies.at[pl.ds(base + c * _CH, _CH)], midx.at[c])
    pltpu.sync_copy(w, wv)
    pltpu.sync_copy(b, bv)

    for c in range(_NCHUNK):
        for g in range(_GPC):
            o = g * _L
            uq.at[c][pl.ds(o, _L)] = jnp.right_shift(uidx.at[c][pl.ds(o, _L)], 2)
            mq.at[c][pl.ds(o, _L)] = jnp.right_shift(midx.at[c][pl.ds(o, _L)], 2)

    iota = lax.iota(jnp.int32, _L)
    bias = bv[...]

    bufs = [(urows0, mrows0, sem0), (urows1, mrows1, sem1)]

    def fire(c):
        ub, mb, s = bufs[c % 2]
        return (pltpu.async_copy(utab.at[uq.at[c]], ub, s),
                pltpu.async_copy(mtab.at[mq.at[c]], mb, s))

    pend = {0: fire(0), 1: fire(1)}

    for c in range(_NCHUNK):
        urows, mrows, _ = bufs[c % 2]
        cp_u, cp_m = pend.pop(c)
        cp_u.wait()
        cp_m.wait()

        def group_body(g, carry, urows=urows, mrows=mrows):
            o = g * _L
            uv = uidx.at[c][pl.ds(o, _L)]
            mv = midx.at[c][pl.ds(o, _L)]
            rows = o + iota
            ucol = jnp.left_shift(jnp.bitwise_and(uv, 3), 5)
            mcol = jnp.left_shift(jnp.bitwise_and(mv, 3), 5)
            acc0 = plsc.load_gather(urows, [rows, ucol]) \
                * plsc.load_gather(mrows, [rows, mcol]) * wv[0]
            acc1 = plsc.load_gather(urows, [rows, ucol + 1]) \
                * plsc.load_gather(mrows, [rows, mcol + 1]) * wv[1]
            for d in range(2, _D, 2):
                acc0 = acc0 + plsc.load_gather(urows, [rows, ucol + d]) \
                    * plsc.load_gather(mrows, [rows, mcol + d]) * wv[d]
                acc1 = acc1 + plsc.load_gather(urows, [rows, ucol + d + 1]) \
                    * plsc.load_gather(mrows, [rows, mcol + d + 1]) * wv[d + 1]
            x = acc0 + acc1 + bias
            y = 1.0 / (1.0 + jnp.exp(-x))
            outv[pl.ds(c * _CH + o, _L)] = y
            return carry

        lax.fori_loop(0, _GPC, group_body, 0)
        if c + 2 < _NCHUNK:
            pend[c + 2] = fire(c + 2)

    pltpu.sync_copy(outv, out.at[pl.ds(base, _BPW)])


def kernel(users, movies, user_table, movie_table, lin_w, lin_b):
    mesh = plsc.VectorSubcoreMesh(core_axis_name="c", subcore_axis_name="s")
    f = pl.kernel(
        _gmf_body,
        mesh=mesh,
        compiler_params=pltpu.CompilerParams(needs_layout_passes=False),
        out_type=jax.ShapeDtypeStruct((_B,), jnp.float32),
        scratch_types=[
            pltpu.VMEM((_NCHUNK, _CH), jnp.int32),
            pltpu.VMEM((_NCHUNK, _CH), jnp.int32),
            pltpu.VMEM((_NCHUNK, _CH), jnp.int32),
            pltpu.VMEM((_NCHUNK, _CH), jnp.int32),
            pltpu.VMEM((_CH, 128), jnp.float32),
            pltpu.VMEM((_CH, 128), jnp.float32),
            pltpu.VMEM((_CH, 128), jnp.float32),
            pltpu.VMEM((_CH, 128), jnp.float32),
            pltpu.VMEM((_D, _L), jnp.float32),
            pltpu.VMEM((_L,), jnp.float32),
            pltpu.VMEM((_BPW,), jnp.float32),
            pltpu.SemaphoreType.DMA,
            pltpu.SemaphoreType.DMA,
        ],
    )
    wb = jnp.broadcast_to(
        lin_w.astype(jnp.float32).reshape(_D, 1), (_D, _L))
    bb = jnp.broadcast_to(lin_b.astype(jnp.float32).reshape(()), (_L,))
    out = f(users.astype(jnp.int32), movies.astype(jnp.int32),
            user_table.reshape(-1, 128), movie_table.reshape(-1, 128),
            wb, bb)
    return out.reshape(_B, 1)
